# Initial kernel scaffold; baseline (speedup 1.0000x reference)
import jax
import jax.numpy as jnp
from jax.experimental import pallas as pl


def kernel(batch, table):
    raise NotImplementedError("write your pallas kernel here")


# speedup vs baseline: 2.1148x; 2.1148x over previous
import jax
import jax.numpy as jnp
from jax.experimental import pallas as pl
from jax.experimental.pallas import tpu as pltpu
from jax.experimental.pallas import tpu_sc as plsc

EMBED_DIM = 32
WINDOW = 128


def kernel(batch, table):
    n_rows, seq = batch.shape
    num_indices = n_rows * seq
    idx = batch.reshape(1, num_indices)

    @pl.kernel(
        out_type=jax.ShapeDtypeStruct((num_indices, EMBED_DIM), table.dtype),
        mesh=plsc.VectorSubcoreMesh(core_axis_name="c", subcore_axis_name="s"),
        compiler_params=pltpu.CompilerParams(use_tc_tiling_on_sc=False),
    )
    def sc_gather(table_hbm, idx_hbm, out_hbm):
        def body(idx_vmem, out_vmem):
            pltpu.sync_copy(table_hbm.at[idx_vmem.at[0]], out_vmem)

        pltpu.emit_pipeline(
            body,
            grid=(num_indices // WINDOW,),
            in_specs=[pl.BlockSpec((1, WINDOW), lambda i: (0, i))],
            out_specs=[pl.BlockSpec((WINDOW, EMBED_DIM), lambda i: (i, 0))],
            core_axis_name=("c", "s"),
            dimension_semantics=(pltpu.PARALLEL,),
        )(idx_hbm, out_hbm)

    out = sc_gather(table, idx)
    return out.reshape(n_rows, seq, EMBED_DIM)
